# baseline (speedup 1.0000x reference)
import dataclasses
import functools
import jax
import jax.numpy as jnp
from jax import lax
from jax.experimental import pallas as pl
from jax.experimental.pallas import tpu as pltpu
from jax.experimental.pallas import tpu_sc as plsc

_N_PROTO = 8192
_FEAT = 1024
_SHARED = 256
_TOPK = 32
_TEMP = 0.1
_BATCH = 8192
_BLK_R = 256
_NCHUNK = 64
_LPC = 128


def _a_body(f_ref, w_ref, p_ref, sims_ref, m64_ref, m0_ref):
    f = f_ref[...]
    w = w_ref[...]
    proj = lax.dot_general(f, w, (((1,), (1,)), ((), ())),
                           preferred_element_type=jnp.float32)
    pn = jnp.sqrt(jnp.sum(proj * proj, axis=1, keepdims=True))
    proj = proj / jnp.maximum(pn, 1e-12)
    sims = lax.dot_general(proj, p_ref[...], (((1,), (1,)), ((), ())),
                           preferred_element_type=jnp.float32)
    sims_ref[...] = sims
    m64_ref[...] = jnp.max(sims.reshape(_BLK_R, _NCHUNK, _LPC), axis=2)
    m0_ref[...] = jnp.max(sims, axis=1, keepdims=True)


def _b_body(sims_ref, v32_ref, m0_ref, p_ref, o_ref):
    sims = sims_ref[...]
    v32 = v32_ref[...]
    m0 = m0_ref[...]
    act = jnp.where(sims >= v32, jnp.exp((sims - m0) * (1.0 / _TEMP)), 0.0)
    denom = jnp.sum(act, axis=1, keepdims=True)
    emb = lax.dot_general(act.astype(jnp.bfloat16),
                          p_ref[...].astype(jnp.bfloat16),
                          (((1,), (0,)), ((), ())),
                          preferred_element_type=jnp.float32)
    emb = emb / denom
    en = jnp.sqrt(jnp.sum(emb * emb, axis=1, keepdims=True))
    o_ref[...] = emb / jnp.maximum(en, 1e-12)


def _sortd(k, v):
    ks, vs = plsc.sort_key_val(k, v, descending=True)
    return ks, vs


def _merge16(ak, av, bk, bv):
    rbk = lax.rev(bk, (0,))
    rbv = lax.rev(bv, (0,))
    m = ak >= rbk
    lk = jnp.where(m, ak, rbk)
    lv = jnp.where(m, av, rbv)
    hk = jnp.where(m, rbk, ak)
    hv = jnp.where(m, rbv, av)
    lk, lv = _sortd(lk, lv)
    hk, hv = _sortd(hk, hv)
    return lk, lv, hk, hv


def _merge32_top32(a1k, a1v, a2k, a2v, b1k, b1v, b2k, b2v):
    rb1k, rb1v = lax.rev(b1k, (0,)), lax.rev(b1v, (0,))
    rb2k, rb2v = lax.rev(b2k, (0,)), lax.rev(b2v, (0,))
    m1 = a1k >= rb2k
    l1k = jnp.where(m1, a1k, rb2k)
    l1v = jnp.where(m1, a1v, rb2v)
    m2 = a2k >= rb1k
    l2k = jnp.where(m2, a2k, rb1k)
    l2v = jnp.where(m2, a2v, rb1v)
    m3 = l1k >= l2k
    uk = jnp.where(m3, l1k, l2k)
    uv = jnp.where(m3, l1v, l2v)
    vk = jnp.where(m3, l2k, l1k)
    vv = jnp.where(m3, l2v, l1v)
    uk, uv = _sortd(uk, uv)
    vk, vv = _sortd(vk, vv)
    return uk, uv, vk, vv


def _sc_v32(m64, tbl):
    info = plsc.get_sparse_core_info()
    nw = info.num_cores * info.num_subcores
    rows_pw = _BATCH // nw
    ngrp = rows_pw // 8
    cp = pltpu.CompilerParams()
    if "needs_layout_passes" in pltpu.CompilerParams.__dataclass_fields__:
        cp = dataclasses.replace(cp, needs_layout_passes=False)
    mesh = plsc.VectorSubcoreMesh(core_axis_name="c", subcore_axis_name="s")
    it16 = lambda: lax.iota(jnp.int32, 16)

    @functools.partial(
        pl.kernel,
        out_type=jax.ShapeDtypeStruct((_BATCH,), jnp.float32),
        mesh=mesh,
        compiler_params=cp,
        scratch_types=[
            pltpu.VMEM((rows_pw, _NCHUNK), jnp.float32),
            pltpu.VMEM((256,), jnp.int32),
            pltpu.VMEM((256,), jnp.int32),
            pltpu.VMEM((256, _LPC), jnp.float32),
            pltpu.VMEM((256, _LPC), jnp.float32),
            pltpu.VMEM((4224,), jnp.float32),
            pltpu.VMEM((rows_pw,), jnp.float32),
            pltpu.SemaphoreType.DMA,
            pltpu.SemaphoreType.DMA,
            pltpu.SemaphoreType.DMA,
        ],
    )
    def k(m64_hbm, tbl_hbm, out_hbm, mg, gidx0, gidx1, cand0, cand1,
          comp, stage, sem0, sem1, semm):
        wid = lax.axis_index("s") * info.num_cores + lax.axis_index("c")
        base = wid * rows_pw
        neg = jnp.full((16,), -jnp.inf, jnp.float32)
        pltpu.async_copy(m64_hbm.at[pl.ds(base, rows_pw)], mg, semm).wait()

        def phase_a(g, gidx, cand, sem):
            taus = neg
            for s in range(8):
                rl = g * 8 + s
                ka = mg[rl, pl.ds(0, 16)]
                kb = mg[rl, pl.ds(16, 16)]
                kc = mg[rl, pl.ds(32, 16)]
                kd = mg[rl, pl.ds(48, 16)]
                ia = it16()
                ka, va = _sortd(ka, ia)
                kb, vb = _sortd(kb, ia + 16)
                kc, vc = _sortd(kc, ia + 32)
                kd, vd = _sortd(kd, ia + 48)
                a1k, a1v, a2k, a2v = _merge16(ka, va, kb, vb)
                b1k, b1v, b2k, b2v = _merge16(kc, vc, kd, vd)
                t1k, t1v, t2k, t2v = _merge32_top32(
                    a1k, a1v, a2k, a2v, b1k, b1v, b2k, b2v)
                tau = lax.reduce_min(t2k, (0,))
                taus = jnp.where(it16() == s, jnp.full((16,), tau), taus)
                rbase = (base + rl) * _NCHUNK
                gidx[pl.ds(s * 32, 16)] = t1v + rbase
                gidx[pl.ds(s * 32 + 16, 16)] = t2v + rbase
            pltpu.async_copy(tbl_hbm.at[gidx.at[pl.ds(0, 128)]],
                             cand.at[pl.ds(0, 128)], sem)
            pltpu.async_copy(tbl_hbm.at[gidx.at[pl.ds(128, 128)]],
                             cand.at[pl.ds(128, 128)], sem)
            return taus

        def drain(gidx, cand, sem):
            pltpu.make_async_copy(tbl_hbm.at[gidx.at[pl.ds(0, 128)]],
                                  cand.at[pl.ds(0, 128)], sem).wait()
            pltpu.make_async_copy(tbl_hbm.at[gidx.at[pl.ds(128, 128)]],
                                  cand.at[pl.ds(128, 128)], sem).wait()

        def phase_b(g, taus, cand, acc):
            for s in range(8):
                tau = lax.reduce_max(
                    jnp.where(it16() == s, taus, neg), (0,))
                tsp = jnp.full((16,), tau)

                def pb(j, off):
                    crow = s * 32 + lax.shift_right_logical(j, 3)
                    lane = lax.mul(lax.rem(j, 8), 16)
                    v = cand[crow, pl.ds(lane, 16)]
                    mk = v >= tsp
                    plsc.store_compressed(comp.at[pl.ds(off, 16)], v, mask=mk)
                    cnt = lax.reduce_max(
                        plsc.all_reduce_population_count(mk), (0,))
                    return off + cnt

                off = lax.fori_loop(0, 256, pb, 0)
                comp[pl.ds(off, 16)] = neg

                def sel(kk, carry):
                    s1, s2 = carry
                    v = comp[pl.ds(kk * 16, 16)]
                    vs, _ = _sortd(v, v)
                    l2 = jnp.maximum(s2, lax.rev(vs, (0,)))
                    u = jnp.maximum(s1, l2)
                    w = jnp.minimum(s1, l2)
                    u, _ = _sortd(u, u)
                    w, _ = _sortd(w, w)
                    return (u, w)

                nk = lax.div(off + 15, 16)
                _, s2f = lax.fori_loop(0, nk, sel, (neg, neg))
                v32 = lax.reduce_min(s2f, (0,))
                acc = jnp.where(it16() == ((g * 8 + s) % 16),
                                jnp.full((16,), v32), acc)
            return acc

        taus0_init = phase_a(0, gidx0, cand0, sem0)

        def body(i, carry):
            acc, taus0 = carry
            g0 = i * 2
            g1 = g0 + 1
            taus1 = phase_a(g1, gidx1, cand1, sem1)
            drain(gidx0, cand0, sem0)
            acc = phase_b(g0, taus0, cand0, acc)

            def fire_next():
                return phase_a(g0 + 2, gidx0, cand0, sem0)

            taus0n = lax.cond(g0 + 2 < ngrp, fire_next, lambda: neg)
            drain(gidx1, cand1, sem1)
            acc = phase_b(g1, taus1, cand1, acc)
            stage[pl.ds(i * 16, 16)] = acc
            return (acc, taus0n)

        lax.fori_loop(0, ngrp // 2, body, (neg, taus0_init))
        pltpu.sync_copy(stage, out_hbm.at[pl.ds(base, rows_pw)])

    return k(m64, tbl)


@functools.partial(jax.jit, static_argnames=("interpret",))
def kernel(features, W_vis, prototypes, interpret=False):
    grid = (_BATCH // _BLK_R,)
    sims, m64, m0 = pl.pallas_call(
        _a_body,
        grid=grid,
        in_specs=[
            pl.BlockSpec((_BLK_R, _FEAT), lambda i: (i, 0)),
            pl.BlockSpec((_SHARED, _FEAT), lambda i: (0, 0)),
            pl.BlockSpec((_N_PROTO, _SHARED), lambda i: (0, 0)),
        ],
        out_specs=[
            pl.BlockSpec((_BLK_R, _N_PROTO), lambda i: (i, 0)),
            pl.BlockSpec((_BLK_R, _NCHUNK), lambda i: (i, 0)),
            pl.BlockSpec((_BLK_R, 1), lambda i: (i, 0)),
        ],
        out_shape=[
            jax.ShapeDtypeStruct((_BATCH, _N_PROTO), jnp.float32),
            jax.ShapeDtypeStruct((_BATCH, _NCHUNK), jnp.float32),
            jax.ShapeDtypeStruct((_BATCH, 1), jnp.float32),
        ],
        compiler_params=pltpu.CompilerParams(
            dimension_semantics=("parallel",)),
        interpret=interpret,
    )(features, W_vis, prototypes)

    tbl = jnp.reshape(sims, (_BATCH * _NCHUNK, _LPC))
    v32 = _sc_v32(m64, tbl).reshape(_BATCH, 1)

    return pl.pallas_call(
        _b_body,
        grid=grid,
        in_specs=[
            pl.BlockSpec((_BLK_R, _N_PROTO), lambda i: (i, 0)),
            pl.BlockSpec((_BLK_R, 1), lambda i: (i, 0)),
            pl.BlockSpec((_BLK_R, 1), lambda i: (i, 0)),
            pl.BlockSpec((_N_PROTO, _SHARED), lambda i: (0, 0)),
        ],
        out_specs=pl.BlockSpec((_BLK_R, _SHARED), lambda i: (i, 0)),
        out_shape=jax.ShapeDtypeStruct((_BATCH, _SHARED), jnp.float32),
        compiler_params=pltpu.CompilerParams(
            dimension_semantics=("parallel",)),
        interpret=interpret,
    )(sims, v32, m0, prototypes)
